# baseline (speedup 1.0000x reference)
import functools

import jax
import jax.numpy as jnp
from jax import lax
from jax.experimental import pallas as pl
from jax.experimental.pallas import tpu as pltpu
from jax.experimental.pallas import tpu_sc as plsc

K = 2
WEIGHT = 1.0
MARGIN = 0.5

L = 16
NC = 2
NS = 16
NW = NC * NS

TILE_R = 8
TILE_C = 128

CHUNK_H = 208
N_FULL = 60
QUARTER = 12504
NLANES = TILE_C // L


def _merge_pair(m1a, m2a, m1b, m2b):
    m1 = jnp.maximum(m1a, m1b)
    m2 = jnp.maximum(jnp.minimum(m1a, m1b), jnp.maximum(m2a, m2b))
    return m1, m2


def _make_sc_kernel(B, C):
    n_old = C // 2
    rows_per = TILE_C // 4

    tail3 = (C - n_old) - 3 * QUARTER - N_FULL * CHUNK_H
    tail012 = QUARTER - N_FULL * CHUNK_H

    mesh = plsc.VectorSubcoreMesh(core_axis_name="c", subcore_axis_name="s")

    @functools.partial(
        pl.kernel,
        mesh=mesh,
        compiler_params=pltpu.CompilerParams(
            needs_layout_passes=False, skip_device_barrier=True),
        out_type=jax.ShapeDtypeStruct((NW, 2 * L), jnp.float32),
        scratch_types=[
            pltpu.VMEM((rows_per,), jnp.int32),
            pltpu.VMEM((rows_per,), jnp.float32),
            pltpu.VMEM((CHUNK_H, TILE_C), jnp.float32),
            pltpu.VMEM((CHUNK_H, TILE_C), jnp.float32),
            pltpu.VMEM((tail012, TILE_C), jnp.float32),
            pltpu.VMEM((tail3, TILE_C), jnp.float32),
            pltpu.VMEM((rows_per, TILE_R, TILE_C), jnp.float32),
            pltpu.VMEM((2 * TILE_R * L,), jnp.float32),
            pltpu.VMEM((4 * 2 * TILE_R * L,), jnp.float32),
            pltpu.VMEM((2 * L,), jnp.float32),
            pltpu.VMEM_SHARED((NS, 2 * TILE_R * L), jnp.float32),
            pltpu.SemaphoreType.DMA,
            pltpu.SemaphoreType.DMA,
            pltpu.SemaphoreType.DMA,
            pltpu.SemaphoreType.DMA,
            pltpu.SemaphoreType.DMA,
        ],
    )
    def sc_kernel(scoresT_hbm, labels_hbm, hard_hbm, out_hbm,
                  lab_v, hard_v, buf_a, buf_b, buf_ta, buf_tb, gt_v,
                  stage_v, peer_v, res_v, shared,
                  sem_s, sem_a, sem_b, sem_t, sem_g):
        cid = lax.axis_index("c")
        sid = lax.axis_index("s")
        j = cid * 4 + sid // 4
        k = sid % 4
        wid = cid * NS + sid
        row_base = j * TILE_C + k * rows_per
        batch0 = pl.multiple_of(j * TILE_C, TILE_C)
        cls0 = pl.multiple_of(n_old + k * QUARTER, TILE_R)

        pltpu.async_copy(labels_hbm.at[pl.ds(row_base, rows_per)],
                         lab_v, sem_s)
        pltpu.make_async_copy(labels_hbm.at[pl.ds(row_base, rows_per)],
                              lab_v, sem_s).wait()
        pltpu.sync_copy(hard_hbm.at[pl.ds(row_base, rows_per)], hard_v)

        for b in range(rows_per // L):
            lab16 = lab_v[pl.ds(b * L, L)]
            for jj in range(L):
                lab = lab16[jj]
                c8 = pl.multiple_of(
                    (lax.shift_right_logical(lab, 3)) * TILE_R, TILE_R)
                pltpu.async_copy(
                    scoresT_hbm.at[pl.ds(c8, TILE_R),
                                   pl.ds(batch0, TILE_C)],
                    gt_v.at[b * L + jj], sem_g)

        neg = jnp.full((L,), -jnp.inf, jnp.float32)
        zero = jnp.zeros((L,), jnp.float32)

        def chunk_src(c):
            off = pl.multiple_of(cls0 + c * CHUNK_H, TILE_R)
            return scoresT_hbm.at[pl.ds(off, CHUNK_H),
                                  pl.ds(batch0, TILE_C)]

        def accum_chunk(buf, height, acc, unroll=4):
            def it(i, a):
                a1, a2 = a
                n1, n2 = [], []
                for u in range(NLANES):
                    v = buf[i, pl.ds(u * L, L)]
                    n2.append(jnp.maximum(a2[u], jnp.minimum(a1[u], v)))
                    n1.append(jnp.maximum(a1[u], v))
                return tuple(n1), tuple(n2)

            return lax.fori_loop(0, height, it, acc,
                                 unroll=min(unroll, height))

        tail_off = pl.multiple_of(cls0 + N_FULL * CHUNK_H, TILE_R)

        @pl.when(k < 3)
        def _():
            pltpu.async_copy(
                scoresT_hbm.at[pl.ds(tail_off, tail012),
                               pl.ds(batch0, TILE_C)], buf_ta, sem_t)

        @pl.when(k == 3)
        def _():
            pltpu.async_copy(
                scoresT_hbm.at[pl.ds(tail_off, tail3),
                               pl.ds(batch0, TILE_C)], buf_tb, sem_t)

        pltpu.async_copy(chunk_src(0), buf_a, sem_a)
        pltpu.async_copy(chunk_src(1), buf_b, sem_b)

        acc0 = ((neg,) * NLANES, (neg,) * NLANES)

        def pair_body(p, acc):
            c0 = 2 * p
            pltpu.make_async_copy(chunk_src(c0), buf_a, sem_a).wait()

            @pl.when(c0 + 2 < N_FULL)
            def _():
                pltpu.async_copy(chunk_src(c0 + 2), buf_a, sem_a)

            acc = accum_chunk(buf_a, CHUNK_H, acc)

            pltpu.make_async_copy(chunk_src(c0 + 1), buf_b, sem_b).wait()

            @pl.when(c0 + 3 < N_FULL)
            def _():
                pltpu.async_copy(chunk_src(c0 + 3), buf_b, sem_b)

            return accum_chunk(buf_b, CHUNK_H, acc)

        m1s, m2s = lax.fori_loop(0, N_FULL // 2, pair_body, acc0)

        @pl.when(k < 3)
        def _():
            pltpu.make_async_copy(
                scoresT_hbm.at[pl.ds(tail_off, tail012),
                               pl.ds(batch0, TILE_C)], buf_ta, sem_t).wait()

        @pl.when(k == 3)
        def _():
            pltpu.make_async_copy(
                scoresT_hbm.at[pl.ds(tail_off, tail3),
                               pl.ds(batch0, TILE_C)], buf_tb, sem_t).wait()

        tk = jnp.full((L,), k, jnp.int32)
        is3 = tk == 3
        m1a, m2a = accum_chunk(buf_ta, tail012, (m1s, m2s))
        m1b, m2b = accum_chunk(buf_tb, tail3, (m1s, m2s))
        m1s = tuple(jnp.where(is3, b_, a_) for a_, b_ in zip(m1a, m1b))
        m2s = tuple(jnp.where(is3, b_, a_) for a_, b_ in zip(m2a, m2b))

        def gt_drain(r, carry):
            pltpu.make_async_copy(
                scoresT_hbm.at[pl.ds(0, TILE_R), pl.ds(0, TILE_C)],
                gt_v.at[r], sem_g).wait()
            return carry

        lax.fori_loop(0, rows_per, gt_drain, jnp.int32(0))

        for u in range(NLANES):
            stage_v[pl.ds(u * L, L)] = m1s[u]
            stage_v[pl.ds((TILE_R + u) * L, L)] = m2s[u]
        pltpu.sync_copy(stage_v, shared.at[sid])
        plsc.subcore_barrier()

        base_peer = (sid // 4) * 4
        for kk in range(4):
            pltpu.sync_copy(shared.at[base_peer + kk],
                            peer_v.at[pl.ds(kk * 2 * TILE_R * L,
                                            2 * TILE_R * L)])

        iota = lax.iota(jnp.int32, L)
        loss_acc = zero
        hard_acc = zero
        for m in range(2):
            u_mine = 2 * k + m
            mm1 = None
            for kk in range(4):
                o1 = kk * 2 * TILE_R * L + u_mine * L
                o2 = o1 + TILE_R * L
                p1 = plsc.load_gather(peer_v, [o1 + iota])
                p2 = plsc.load_gather(peer_v, [o2 + iota])
                if mm1 is None:
                    mm1, mm2 = p1, p2
                else:
                    mm1, mm2 = _merge_pair(mm1, mm2, p1, p2)

            lab16 = lab_v[pl.ds(m * L, L)]
            hd16 = hard_v[pl.ds(m * L, L)]
            ridx = jnp.full((L,), m * L, jnp.int32) + iota
            coff = lab16 & (TILE_R - 1)
            lane = jnp.full((L,), k * rows_per + m * L, jnp.int32) + iota
            gt16 = plsc.load_gather(gt_v, [ridx, coff, lane])
            pe = (jnp.maximum(mm1 - gt16 + MARGIN, zero)
                  + jnp.maximum(mm2 - gt16 + MARGIN, zero))
            loss_acc = loss_acc + hd16 * pe
            hard_acc = hard_acc + hd16

        res_v[pl.ds(0, L)] = loss_acc
        res_v[pl.ds(L, L)] = hard_acc
        pltpu.sync_copy(res_v, out_hbm.at[wid])

    return sc_kernel


def kernel(scores, labels, num_old_classes):
    B, C = scores.shape
    labels = labels.astype(jnp.int32)
    hard = (labels < num_old_classes).astype(jnp.float32)

    partials = _make_sc_kernel(B, C)(scores.T, labels, hard)
    loss_sum = jnp.sum(partials[:, :L])
    hard_num = jnp.sum(partials[:, L:])
    denom = jnp.maximum(hard_num * K, 1.0)
    return WEIGHT * loss_sum / denom


# speedup vs baseline: 1.0029x; 1.0029x over previous
import functools

import jax
import jax.numpy as jnp
from jax import lax
from jax.experimental import pallas as pl
from jax.experimental.pallas import tpu as pltpu
from jax.experimental.pallas import tpu_sc as plsc

K = 2
WEIGHT = 1.0
MARGIN = 0.5

L = 16
NC = 2
NS = 16
NW = NC * NS

TILE_R = 8
TILE_C = 128

CHUNK_H = 208
N_FULL = 60
QUARTER = 12504
NLANES = TILE_C // L


def _merge_pair(m1a, m2a, m1b, m2b):
    m1 = jnp.maximum(m1a, m1b)
    m2 = jnp.maximum(jnp.minimum(m1a, m1b), jnp.maximum(m2a, m2b))
    return m1, m2


def _make_sc_kernel(B, C):
    n_old = C // 2
    rows_per = TILE_C // 4

    tail3 = (C - n_old) - 3 * QUARTER - N_FULL * CHUNK_H
    tail012 = QUARTER - N_FULL * CHUNK_H

    mesh = plsc.VectorSubcoreMesh(core_axis_name="c", subcore_axis_name="s")

    @functools.partial(
        pl.kernel,
        mesh=mesh,
        compiler_params=pltpu.CompilerParams(needs_layout_passes=False),
        out_type=jax.ShapeDtypeStruct((NW, 2 * L), jnp.float32),
        scratch_types=[
            pltpu.VMEM((rows_per,), jnp.int32),
            pltpu.VMEM((rows_per,), jnp.float32),
            pltpu.VMEM((CHUNK_H, TILE_C), jnp.float32),
            pltpu.VMEM((CHUNK_H, TILE_C), jnp.float32),
            pltpu.VMEM((CHUNK_H, TILE_C), jnp.float32),
            pltpu.VMEM((tail012, TILE_C), jnp.float32),
            pltpu.VMEM((tail3, TILE_C), jnp.float32),
            pltpu.VMEM((rows_per, TILE_R, TILE_C), jnp.float32),
            pltpu.VMEM((2 * TILE_R * L,), jnp.float32),
            pltpu.VMEM((4 * 2 * TILE_R * L,), jnp.float32),
            pltpu.VMEM((2 * L,), jnp.float32),
            pltpu.VMEM_SHARED((NS, 2 * TILE_R * L), jnp.float32),
            pltpu.SemaphoreType.DMA,
            pltpu.SemaphoreType.DMA,
            pltpu.SemaphoreType.DMA,
            pltpu.SemaphoreType.DMA,
            pltpu.SemaphoreType.DMA,
            pltpu.SemaphoreType.DMA,
        ],
    )
    def sc_kernel(scoresT_hbm, labels_hbm, hard_hbm, out_hbm,
                  lab_v, hard_v, buf_a, buf_b, buf_c, buf_ta, buf_tb, gt_v,
                  stage_v, peer_v, res_v, shared,
                  sem_s, sem_a, sem_b, sem_c, sem_t, sem_g):
        cid = lax.axis_index("c")
        sid = lax.axis_index("s")
        j = cid * 4 + sid // 4
        k = sid % 4
        wid = cid * NS + sid
        row_base = j * TILE_C + k * rows_per
        batch0 = pl.multiple_of(j * TILE_C, TILE_C)
        cls0 = pl.multiple_of(n_old + k * QUARTER, TILE_R)

        pltpu.async_copy(labels_hbm.at[pl.ds(row_base, rows_per)],
                         lab_v, sem_s)
        pltpu.make_async_copy(labels_hbm.at[pl.ds(row_base, rows_per)],
                              lab_v, sem_s).wait()
        pltpu.sync_copy(hard_hbm.at[pl.ds(row_base, rows_per)], hard_v)

        for b in range(rows_per // L):
            lab16 = lab_v[pl.ds(b * L, L)]
            for jj in range(L):
                lab = lab16[jj]
                c8 = pl.multiple_of(
                    (lax.shift_right_logical(lab, 3)) * TILE_R, TILE_R)
                pltpu.async_copy(
                    scoresT_hbm.at[pl.ds(c8, TILE_R),
                                   pl.ds(batch0, TILE_C)],
                    gt_v.at[b * L + jj], sem_g)

        neg = jnp.full((L,), -jnp.inf, jnp.float32)
        zero = jnp.zeros((L,), jnp.float32)

        def chunk_src(c):
            off = pl.multiple_of(cls0 + c * CHUNK_H, TILE_R)
            return scoresT_hbm.at[pl.ds(off, CHUNK_H),
                                  pl.ds(batch0, TILE_C)]

        def accum_chunk(buf, height, acc, unroll=4):
            def it(i, a):
                a1, a2 = a
                n1, n2 = [], []
                for u in range(NLANES):
                    v = buf[i, pl.ds(u * L, L)]
                    n2.append(jnp.maximum(a2[u], jnp.minimum(a1[u], v)))
                    n1.append(jnp.maximum(a1[u], v))
                return tuple(n1), tuple(n2)

            return lax.fori_loop(0, height, it, acc,
                                 unroll=min(unroll, height))

        tail_off = pl.multiple_of(cls0 + N_FULL * CHUNK_H, TILE_R)

        @pl.when(k < 3)
        def _():
            pltpu.async_copy(
                scoresT_hbm.at[pl.ds(tail_off, tail012),
                               pl.ds(batch0, TILE_C)], buf_ta, sem_t)

        @pl.when(k == 3)
        def _():
            pltpu.async_copy(
                scoresT_hbm.at[pl.ds(tail_off, tail3),
                               pl.ds(batch0, TILE_C)], buf_tb, sem_t)

        pltpu.async_copy(chunk_src(0), buf_a, sem_a)
        pltpu.async_copy(chunk_src(1), buf_b, sem_b)
        pltpu.async_copy(chunk_src(2), buf_c, sem_c)

        acc0 = ((neg,) * NLANES, (neg,) * NLANES)
        ring = ((buf_a, sem_a), (buf_b, sem_b), (buf_c, sem_c))

        def trip_body(p, acc):
            c0 = 3 * p
            for i, (buf, sem) in enumerate(ring):
                pltpu.make_async_copy(chunk_src(c0 + i), buf, sem).wait()
                acc = accum_chunk(buf, CHUNK_H, acc)

                @pl.when(c0 + i + 3 < N_FULL)
                def _():
                    pltpu.async_copy(chunk_src(c0 + i + 3), buf, sem)

            return acc

        m1s, m2s = lax.fori_loop(0, N_FULL // 3, trip_body, acc0)

        @pl.when(k < 3)
        def _():
            pltpu.make_async_copy(
                scoresT_hbm.at[pl.ds(tail_off, tail012),
                               pl.ds(batch0, TILE_C)], buf_ta, sem_t).wait()

        @pl.when(k == 3)
        def _():
            pltpu.make_async_copy(
                scoresT_hbm.at[pl.ds(tail_off, tail3),
                               pl.ds(batch0, TILE_C)], buf_tb, sem_t).wait()

        tk = jnp.full((L,), k, jnp.int32)
        is3 = tk == 3
        m1a, m2a = accum_chunk(buf_ta, tail012, (m1s, m2s))
        m1b, m2b = accum_chunk(buf_tb, tail3, (m1s, m2s))
        m1s = tuple(jnp.where(is3, b_, a_) for a_, b_ in zip(m1a, m1b))
        m2s = tuple(jnp.where(is3, b_, a_) for a_, b_ in zip(m2a, m2b))

        def gt_drain(r, carry):
            pltpu.make_async_copy(
                scoresT_hbm.at[pl.ds(0, TILE_R), pl.ds(0, TILE_C)],
                gt_v.at[r], sem_g).wait()
            return carry

        lax.fori_loop(0, rows_per, gt_drain, jnp.int32(0))

        for u in range(NLANES):
            stage_v[pl.ds(u * L, L)] = m1s[u]
            stage_v[pl.ds((TILE_R + u) * L, L)] = m2s[u]
        pltpu.sync_copy(stage_v, shared.at[sid])
        plsc.subcore_barrier()

        base_peer = (sid // 4) * 4
        for kk in range(4):
            pltpu.sync_copy(shared.at[base_peer + kk],
                            peer_v.at[pl.ds(kk * 2 * TILE_R * L,
                                            2 * TILE_R * L)])

        iota = lax.iota(jnp.int32, L)
        loss_acc = zero
        hard_acc = zero
        for m in range(2):
            u_mine = 2 * k + m
            mm1 = None
            for kk in range(4):
                o1 = kk * 2 * TILE_R * L + u_mine * L
                o2 = o1 + TILE_R * L
                p1 = plsc.load_gather(peer_v, [o1 + iota])
                p2 = plsc.load_gather(peer_v, [o2 + iota])
                if mm1 is None:
                    mm1, mm2 = p1, p2
                else:
                    mm1, mm2 = _merge_pair(mm1, mm2, p1, p2)

            lab16 = lab_v[pl.ds(m * L, L)]
            hd16 = hard_v[pl.ds(m * L, L)]
            ridx = jnp.full((L,), m * L, jnp.int32) + iota
            coff = lab16 & (TILE_R - 1)
            lane = jnp.full((L,), k * rows_per + m * L, jnp.int32) + iota
            gt16 = plsc.load_gather(gt_v, [ridx, coff, lane])
            pe = (jnp.maximum(mm1 - gt16 + MARGIN, zero)
                  + jnp.maximum(mm2 - gt16 + MARGIN, zero))
            loss_acc = loss_acc + hd16 * pe
            hard_acc = hard_acc + hd16

        res_v[pl.ds(0, L)] = loss_acc
        res_v[pl.ds(L, L)] = hard_acc
        pltpu.sync_copy(res_v, out_hbm.at[wid])

    return sc_kernel


def kernel(scores, labels, num_old_classes):
    B, C = scores.shape
    labels = labels.astype(jnp.int32)
    hard = (labels < num_old_classes).astype(jnp.float32)

    partials = _make_sc_kernel(B, C)(scores.T, labels, hard)
    loss_sum = jnp.sum(partials[:, :L])
    hard_num = jnp.sum(partials[:, L:])
    denom = jnp.maximum(hard_num * K, 1.0)
    return WEIGHT * loss_sum / denom
